# Initial kernel scaffold; baseline (speedup 1.0000x reference)
import jax
import jax.numpy as jnp
from jax.experimental import pallas as pl


def kernel(input_ids, token_embeddings, position_embeddings):
    raise NotImplementedError("write your pallas kernel here")


# speedup vs baseline: 1.1276x; 1.1276x over previous
import functools

import jax
import jax.numpy as jnp
from jax import lax
from jax.experimental import pallas as pl
from jax.experimental.pallas import tpu as pltpu
from jax.experimental.pallas import tpu_sc as plsc

NC = 2
NS = 16
NW = NC * NS
L = 16


@functools.lru_cache(maxsize=None)
def _make_kernel(B, S, V, D, C):
    s_per_w = S // NW
    chunks = s_per_w // C
    ncol = D // L

    mesh = plsc.VectorSubcoreMesh(core_axis_name="c", subcore_axis_name="s")

    @functools.partial(
        pl.kernel,
        mesh=mesh,
        out_type=jax.ShapeDtypeStruct((B * S, D), jnp.float32),
        scratch_types=[
            pltpu.VMEM((B, chunks, C), jnp.int32),
            pltpu.VMEM((C, D), jnp.float32),
            pltpu.VMEM((C, D), jnp.float32),
            pltpu.SemaphoreType.DMA,
        ],
    )
    def emb_kernel(ids_hbm, tok_hbm, pos_hbm, out_hbm, idx_v, tokbuf, posbuf, sem):
        wid = lax.axis_index("s") * NC + lax.axis_index("c")
        s0 = wid * s_per_w

        for b in range(B):
            for k in range(chunks):
                pltpu.sync_copy(
                    ids_hbm.at[pl.ds(b * S + s0 + k * C, C)],
                    idx_v.at[b, k],
                )

        def chunk_body(k, carry):
            row0 = s0 + k * C
            pltpu.sync_copy(pos_hbm.at[pl.ds(row0, C)], posbuf)
            for b in range(B):
                pltpu.async_copy(
                    tok_hbm.at[idx_v.at[b, k]], tokbuf, sem
                ).wait()

                def row_body(r, c2):
                    for c in range(ncol):
                        sl = pl.ds(c * L, L)
                        tokbuf[r, sl] = tokbuf[r, sl] + posbuf[r, sl]
                    return c2

                lax.fori_loop(0, C, row_body, 0)
                pltpu.sync_copy(tokbuf, out_hbm.at[pl.ds(b * S + row0, C)])
            return carry

        lax.fori_loop(0, chunks, chunk_body, 0)

    return emb_kernel


def kernel(input_ids, token_embeddings, position_embeddings):
    B, S = input_ids.shape
    V, D = token_embeddings.shape
    ids = input_ids.reshape(-1).astype(jnp.int32)
    k = _make_kernel(B, S, V, D, 64)
    out = k(ids, token_embeddings, position_embeddings)
    return out.reshape(B, S, D)


# speedup vs baseline: 1.4140x; 1.2540x over previous
import functools

import jax
import jax.numpy as jnp
from jax import lax
from jax.experimental import pallas as pl
from jax.experimental.pallas import tpu as pltpu
from jax.experimental.pallas import tpu_sc as plsc

NC = 2
NS = 16
NW = NC * NS
L = 16


@functools.lru_cache(maxsize=None)
def _make_kernel(B, S, V, D, C):
    s_per_w = S // NW
    chunks = s_per_w // C
    ncol = D // L

    mesh = plsc.VectorSubcoreMesh(core_axis_name="c", subcore_axis_name="s")

    @functools.partial(
        pl.kernel,
        mesh=mesh,
        out_type=jax.ShapeDtypeStruct((B * S, D), jnp.float32),
        scratch_types=[
            pltpu.VMEM((B, s_per_w), jnp.int32),
            pltpu.VMEM((B, C, D), jnp.float32),
            pltpu.VMEM((2, C, D), jnp.float32),
            pltpu.SemaphoreType.DMA((B,)),
            pltpu.SemaphoreType.DMA((B,)),
            pltpu.SemaphoreType.DMA((2,)),
        ],
    )
    def emb_kernel(ids_hbm, tok_hbm, pos_hbm, out_hbm, idx_v, tokb, posb, gsem, ssem, psem):
        wid = lax.axis_index("s") * NC + lax.axis_index("c")
        s0 = wid * s_per_w

        for b in range(B):
            pltpu.sync_copy(ids_hbm.at[pl.ds(b * S + s0, s_per_w)], idx_v.at[b])

        def gather(k, b):
            pltpu.async_copy(
                tok_hbm.at[idx_v.at[b, pl.ds(k * C, C)]], tokb.at[b], gsem.at[b]
            )

        def gather_wait(b):
            pltpu.make_async_copy(
                tok_hbm.at[pl.ds(0, C)], tokb.at[b], gsem.at[b]
            ).wait()

        def scatter(k, b):
            pltpu.async_copy(
                tokb.at[b], out_hbm.at[pl.ds(b * S + s0 + k * C, C)], ssem.at[b]
            )

        def scatter_wait(b):
            pltpu.make_async_copy(
                tokb.at[b], out_hbm.at[pl.ds(b * S + s0, C)], ssem.at[b]
            ).wait()

        def pos_load(k, pb):
            pltpu.async_copy(pos_hbm.at[pl.ds(s0 + k * C, C)], posb.at[pb], psem.at[pb])

        def pos_wait(pb):
            pltpu.make_async_copy(
                pos_hbm.at[pl.ds(s0, C)], posb.at[pb], psem.at[pb]
            ).wait()

        pos_load(0, 0)
        gather(0, 0)
        gather(0, 1)

        def outer(i, carry):
            for kk in range(2):
                k = i * 2 + kk
                pb = kk
                pos_wait(pb)
                if kk == 0:
                    pos_load(k + 1, 1 - pb)
                else:
                    @pl.when(i < chunks // 2 - 1)
                    def _():
                        pos_load(k + 1, 1 - pb)

                for b in range(B):
                    gather_wait(b)

                    def row_body(r, c2):
                        for c in range(ncol):
                            sl = pl.ds(c * L, L)
                            tokb[b, r, sl] = tokb[b, r, sl] + posb[pb, r, sl]
                        return c2

                    lax.fori_loop(0, C, row_body, 0)
                    scatter(k, b)

                    b2 = (b + 2) % B
                    k2 = k + (b + 2) // B
                    if b < 2:
                        if kk == 0:
                            @pl.when(i >= 1)
                            def _():
                                scatter_wait(b2)
                        else:
                            scatter_wait(b2)
                        gather(k2, b2)
                    else:
                        if kk == 0:
                            scatter_wait(b2)
                            gather(k2, b2)
                        else:
                            @pl.when(i < chunks // 2 - 1)
                            def _():
                                scatter_wait(b2)
                                gather(k2, b2)
            return carry

        lax.fori_loop(0, chunks // 2, outer, 0)

        for b in range(B):
            scatter_wait(b)

    return emb_kernel


def kernel(input_ids, token_embeddings, position_embeddings):
    B, S = input_ids.shape
    V, D = token_embeddings.shape
    ids = input_ids.reshape(-1).astype(jnp.int32)
    k = _make_kernel(B, S, V, D, 16)
    out = k(ids, token_embeddings, position_embeddings)
    return out.reshape(B, S, D)
